# baseline (speedup 1.0000x reference)
import functools

import jax
import jax.numpy as jnp
from jax import lax
from jax.experimental import pallas as pl
from jax.experimental.pallas import tpu as pltpu
from jax.experimental.pallas import tpu_sc as plsc

B = 16384
D = 32
NA = 1000000
NT = 1000
NC, NS = 2, 16
NW = NC * NS
PER_W = B // NW
PACK = 4
ROWS_W = PER_W // PACK

_mesh = plsc.VectorSubcoreMesh(core_axis_name="c", subcore_axis_name="s")


NPRJ = 16384


def _project_body(tT_ref, v_ref, o_ref):
    o_ref[...] = jax.lax.dot_general(
        tT_ref[...].astype(jnp.bfloat16), v_ref[...].astype(jnp.bfloat16),
        (((0,), (0,)), ((), ())),
        preferred_element_type=jnp.float32)


def _project(tT, vcat, n_rows):
    blk = NPRJ if n_rows >= NPRJ else n_rows
    return pl.pallas_call(
        _project_body,
        grid=(pl.cdiv(n_rows, blk),),
        in_specs=[
            pl.BlockSpec((D, blk), lambda i: (0, i)),
            pl.BlockSpec((D, 128), lambda i: (0, 0)),
        ],
        out_specs=pl.BlockSpec((blk, 128), lambda i: (i, 0)),
        out_shape=jax.ShapeDtypeStruct((n_rows, 128), jnp.float32),
    )(tT, vcat)


def _gather_sum(tab, idx, n_slots, lane0s):
    @functools.partial(
        pl.kernel,
        out_type=jax.ShapeDtypeStruct((B // PACK, 128), jnp.float32),
        mesh=_mesh,
        scratch_types=[
            pltpu.VMEM((PER_W,), jnp.int32),
            pltpu.VMEM((PER_W, 128), jnp.float32),
            pltpu.VMEM((ROWS_W, 128), jnp.float32),
            pltpu.SemaphoreType.DMA,
        ],
        compiler_params=pltpu.CompilerParams(use_tc_tiling_on_sc=False),
    )
    def k(tab_hbm, idx_hbm, out, idx_v, big_v, acc_v, sem):
        wid = lax.axis_index("s") * NC + lax.axis_index("c")
        base = wid * PER_W
        obase = wid * ROWS_W

        for j in range(n_slots):
            pltpu.sync_copy(idx_hbm.at[j].at[pl.ds(base, PER_W)], idx_v)
            pltpu.async_copy(tab_hbm.at[idx_v], big_v, sem).wait()
            lane0 = lane0s[j]
            first = j == 0

            @pl.loop(0, PER_W)
            def _(bb):
                r = bb >> 2
                k_ = bb & 3
                lo = big_v[bb, pl.ds(lane0, 16)]
                hi = big_v[bb, pl.ds(lane0 + 16, 16)]
                if first:
                    acc_v[r, pl.ds(k_ * 32, 16)] = lo
                    acc_v[r, pl.ds(k_ * 32 + 16, 16)] = hi
                else:
                    acc_v[r, pl.ds(k_ * 32, 16)] += lo
                    acc_v[r, pl.ds(k_ * 32 + 16, 16)] += hi

        pltpu.sync_copy(acc_v, out.at[pl.ds(obase, ROWS_W)])

    return k(tab, idx)


def kernel(type_ids, ability_ids, type_table, ability_table, W, b):
    t_idx = type_ids.T.astype(jnp.int32)
    a_idx = ability_ids.T.astype(jnp.int32)

    wt = W.T
    vcat_a = jnp.concatenate([wt[(2 + j) * D:(3 + j) * D, :]
                              for j in range(4)], axis=1)
    vcat_t = jnp.concatenate(
        [wt[0:D, :], wt[D:2 * D, :], jnp.zeros((D, 64), W.dtype)], axis=1)

    tp_tab = _project(type_table.T, vcat_t, NT)
    bias_row = jnp.concatenate([b, jnp.zeros((96,), b.dtype)]).reshape(1, 128)
    tp_tab = tp_tab + bias_row
    vcat_a = vcat_a + 0.0 * tp_tab[0, 0]
    p_tab = _project(ability_table.T, vcat_a, NA)

    out_t = _gather_sum(tp_tab, t_idx, 2, (0, 32))
    out_a = _gather_sum(p_tab, a_idx, 4, (0, 32, 64, 96))
    return (out_t + out_a).reshape(B, D)


# speedup vs baseline: 1.0361x; 1.0361x over previous
import functools

import jax
import jax.numpy as jnp
from jax import lax
from jax.experimental import pallas as pl
from jax.experimental.pallas import tpu as pltpu
from jax.experimental.pallas import tpu_sc as plsc

B = 16384
D = 32
NA = 1000000
NT = 1000
NC, NS = 2, 16
NW = NC * NS
PER_W = B // NW
PACK = 4
ROWS_W = PER_W // PACK

_mesh = plsc.VectorSubcoreMesh(core_axis_name="c", subcore_axis_name="s")


NPRJ = 32768


def _project_body(tT_ref, v_ref, o_ref):
    o_ref[...] = jax.lax.dot_general(
        tT_ref[...].astype(jnp.bfloat16), v_ref[...].astype(jnp.bfloat16),
        (((0,), (0,)), ((), ())),
        preferred_element_type=jnp.float32)


def _project(tT, vcat, n_rows):
    blk = NPRJ if n_rows >= NPRJ else n_rows
    return pl.pallas_call(
        _project_body,
        grid=(pl.cdiv(n_rows, blk),),
        in_specs=[
            pl.BlockSpec((D, blk), lambda i: (0, i)),
            pl.BlockSpec((D, 128), lambda i: (0, 0)),
        ],
        out_specs=pl.BlockSpec((blk, 128), lambda i: (i, 0)),
        out_shape=jax.ShapeDtypeStruct((n_rows, 128), jnp.float32),
    )(tT, vcat)


def _gather_sum(tab, idx, n_slots, lane0s):
    @functools.partial(
        pl.kernel,
        out_type=jax.ShapeDtypeStruct((B // PACK, 128), jnp.float32),
        mesh=_mesh,
        scratch_types=[
            pltpu.VMEM((PER_W,), jnp.int32),
            pltpu.VMEM((PER_W, 128), jnp.float32),
            pltpu.VMEM((ROWS_W, 128), jnp.float32),
            pltpu.SemaphoreType.DMA,
        ],
        compiler_params=pltpu.CompilerParams(use_tc_tiling_on_sc=False),
    )
    def k(tab_hbm, idx_hbm, out, idx_v, big_v, acc_v, sem):
        wid = lax.axis_index("s") * NC + lax.axis_index("c")
        base = wid * PER_W
        obase = wid * ROWS_W

        for j in range(n_slots):
            pltpu.sync_copy(idx_hbm.at[j].at[pl.ds(base, PER_W)], idx_v)
            pltpu.async_copy(tab_hbm.at[idx_v], big_v, sem).wait()
            lane0 = lane0s[j]
            first = j == 0

            @pl.loop(0, PER_W)
            def _(bb):
                r = bb >> 2
                k_ = bb & 3
                lo = big_v[bb, pl.ds(lane0, 16)]
                hi = big_v[bb, pl.ds(lane0 + 16, 16)]
                if first:
                    acc_v[r, pl.ds(k_ * 32, 16)] = lo
                    acc_v[r, pl.ds(k_ * 32 + 16, 16)] = hi
                else:
                    acc_v[r, pl.ds(k_ * 32, 16)] += lo
                    acc_v[r, pl.ds(k_ * 32 + 16, 16)] += hi

        pltpu.sync_copy(acc_v, out.at[pl.ds(obase, ROWS_W)])

    return k(tab, idx)


def kernel(type_ids, ability_ids, type_table, ability_table, W, b):
    t_idx = type_ids.T.astype(jnp.int32)
    a_idx = ability_ids.T.astype(jnp.int32)

    wt = W.T
    vcat_a = jnp.concatenate([wt[(2 + j) * D:(3 + j) * D, :]
                              for j in range(4)], axis=1)
    vcat_t = jnp.concatenate(
        [wt[0:D, :], wt[D:2 * D, :], jnp.zeros((D, 64), W.dtype)], axis=1)

    tp_tab = _project(type_table.T, vcat_t, NT)
    bias_row = jnp.concatenate([b, jnp.zeros((96,), b.dtype)]).reshape(1, 128)
    tp_tab = tp_tab + bias_row
    p_tab = _project(ability_table.T, vcat_a, NA)

    out_t = _gather_sum(tp_tab, t_idx, 2, (0, 32))
    out_a = _gather_sum(p_tab, a_idx, 4, (0, 32, 64, 96))
    return (out_t + out_a).reshape(B, D)


# speedup vs baseline: 1.0871x; 1.0492x over previous
import functools

import jax
import jax.numpy as jnp
from jax import lax
from jax.experimental import pallas as pl
from jax.experimental.pallas import tpu as pltpu
from jax.experimental.pallas import tpu_sc as plsc

B = 16384
D = 32
NA = 1000000
NT = 1000
NC, NS = 2, 16
NW = NC * NS
PER_W = B // NW
PACK = 4
ROWS_W = PER_W // PACK

_mesh = plsc.VectorSubcoreMesh(core_axis_name="c", subcore_axis_name="s")


NPRJ = 32768


def _project_body(tT_ref, v_ref, o_ref):
    o_ref[...] = jax.lax.dot_general(
        tT_ref[...].astype(jnp.bfloat16), v_ref[...].astype(jnp.bfloat16),
        (((0,), (0,)), ((), ())),
        preferred_element_type=jnp.float32)


def _project(tT, vcat, n_rows):
    blk = NPRJ if n_rows >= NPRJ else n_rows
    return pl.pallas_call(
        _project_body,
        grid=(pl.cdiv(n_rows, blk),),
        in_specs=[
            pl.BlockSpec((D, blk), lambda i: (0, i)),
            pl.BlockSpec((D, 128), lambda i: (0, 0)),
        ],
        out_specs=pl.BlockSpec((blk, 128), lambda i: (i, 0)),
        out_shape=jax.ShapeDtypeStruct((n_rows, 128), jnp.float32),
    )(tT, vcat)


CH = 256


def _gather_sum(tab, idx, n_slots, lane0s):
    nch = n_slots * PER_W // CH

    @functools.partial(
        pl.kernel,
        out_type=jax.ShapeDtypeStruct((B // PACK, 128), jnp.float32),
        mesh=_mesh,
        scratch_types=[
            pltpu.VMEM((n_slots * PER_W,), jnp.int32),
            pltpu.VMEM((CH, 128), jnp.float32),
            pltpu.VMEM((CH, 128), jnp.float32),
            pltpu.VMEM((ROWS_W, 128), jnp.float32),
            pltpu.SemaphoreType.DMA,
            pltpu.SemaphoreType.DMA,
            pltpu.SemaphoreType.DMA,
        ],
        compiler_params=pltpu.CompilerParams(use_tc_tiling_on_sc=False),
    )
    def k(tab_hbm, idx_hbm, out, idx_all, big0, big1, acc_v,
          sem0, sem1, semi):
        wid = lax.axis_index("s") * NC + lax.axis_index("c")
        base = wid * PER_W
        obase = wid * ROWS_W
        bigs = (big0, big1)
        sems = (sem0, sem1)

        idx_cps = [
            pltpu.async_copy(idx_hbm.at[j].at[pl.ds(base, PER_W)],
                             idx_all.at[pl.ds(j * PER_W, PER_W)], semi)
            for j in range(n_slots)
        ]
        for cp in idx_cps:
            cp.wait()

        def fire(c):
            return pltpu.async_copy(
                tab_hbm.at[idx_all.at[pl.ds(c * CH, CH)]],
                bigs[c % 2], sems[c % 2])

        handles = [fire(0)]
        for c in range(nch):
            if c + 1 < nch:
                handles.append(fire(c + 1))
            handles[c].wait()
            j = c * CH // PER_W
            boff = (c * CH) % PER_W
            lane0 = lane0s[j]
            first = c * CH < PER_W
            big_v = bigs[c % 2]

            @pl.loop(0, CH)
            def _(i):
                bb = boff + i
                r = bb >> 2
                k_ = bb & 3
                lo = big_v[i, pl.ds(lane0, 16)]
                hi = big_v[i, pl.ds(lane0 + 16, 16)]
                if first:
                    acc_v[r, pl.ds(k_ * 32, 16)] = lo
                    acc_v[r, pl.ds(k_ * 32 + 16, 16)] = hi
                else:
                    acc_v[r, pl.ds(k_ * 32, 16)] += lo
                    acc_v[r, pl.ds(k_ * 32 + 16, 16)] += hi

        pltpu.sync_copy(acc_v, out.at[pl.ds(obase, ROWS_W)])

    return k(tab, idx)


def kernel(type_ids, ability_ids, type_table, ability_table, W, b):
    t_idx = type_ids.T.astype(jnp.int32)
    a_idx = ability_ids.T.astype(jnp.int32)

    wt = W.T
    vcat_a = jnp.concatenate([wt[(2 + j) * D:(3 + j) * D, :]
                              for j in range(4)], axis=1)
    vcat_t = jnp.concatenate(
        [wt[0:D, :], wt[D:2 * D, :], jnp.zeros((D, 64), W.dtype)], axis=1)

    tp_tab = _project(type_table.T, vcat_t, NT)
    bias_row = jnp.concatenate([b, jnp.zeros((96,), b.dtype)]).reshape(1, 128)
    tp_tab = tp_tab + bias_row
    p_tab = _project(ability_table.T, vcat_a, NA)

    out_t = _gather_sum(tp_tab, t_idx, 2, (0, 32))
    out_a = _gather_sum(p_tab, a_idx, 4, (0, 32, 64, 96))
    return (out_t + out_a).reshape(B, D)
